# Initial kernel scaffold; baseline (speedup 1.0000x reference)
import jax
import jax.numpy as jnp
from jax.experimental import pallas as pl


def kernel(token_ids, embedding_matrix):
    raise NotImplementedError("write your pallas kernel here")


# speedup vs baseline: 1.0944x; 1.0944x over previous
import functools

import jax
import jax.numpy as jnp
from jax import lax
from jax.experimental import pallas as pl
from jax.experimental.pallas import tpu as pltpu
from jax.experimental.pallas import tpu_sc as plsc

NUM_WORKERS = 32
CHUNK = 1024


@functools.cache
def _build(B, V, D):
    b_per_w = B // NUM_WORKERS
    n_chunks = b_per_w // CHUNK
    mesh = plsc.VectorSubcoreMesh(core_axis_name="c", subcore_axis_name="s")

    @functools.partial(
        pl.kernel,
        out_type=jax.ShapeDtypeStruct((B, D), jnp.float32),
        mesh=mesh,
        scratch_types=[
            pltpu.VMEM((CHUNK,), jnp.int32),
            pltpu.VMEM((CHUNK, D), jnp.float32),
            pltpu.SemaphoreType.DMA,
        ],
        compiler_params=pltpu.CompilerParams(use_tc_tiling_on_sc=False),
    )
    def gather_kernel(table_hbm, idx_hbm, out_hbm, idx_v, rows_v, sem):
        wid = lax.axis_index("s") * 2 + lax.axis_index("c")
        base = wid * b_per_w

        @pl.loop(0, n_chunks)
        def _chunk(g):
            off = base + g * CHUNK
            pltpu.sync_copy(idx_hbm.at[pl.ds(off, CHUNK)], idx_v)
            pltpu.async_copy(table_hbm.at[idx_v], rows_v, sem).wait()
            pltpu.sync_copy(rows_v, out_hbm.at[pl.ds(off, CHUNK)])

    return gather_kernel


def kernel(token_ids, embedding_matrix):
    S, T = token_ids.shape
    V, D = embedding_matrix.shape
    B = S * T
    flat_ids = token_ids.reshape(B).astype(jnp.int32)
    out = _build(B, V, D)(embedding_matrix, flat_ids)
    return out.reshape(S, T, D)


# speedup vs baseline: 1.1098x; 1.0141x over previous
import functools

import jax
import jax.numpy as jnp
from jax import lax
from jax.experimental import pallas as pl
from jax.experimental.pallas import tpu as pltpu
from jax.experimental.pallas import tpu_sc as plsc

NUM_WORKERS = 32
CHUNK = 640
NBUF = 4


@functools.cache
def _build(B, V, D):
    b_per_w = B // NUM_WORKERS
    n_chunks = b_per_w // CHUNK
    n_groups = n_chunks // NBUF
    mesh = plsc.VectorSubcoreMesh(core_axis_name="c", subcore_axis_name="s")

    @functools.partial(
        pl.kernel,
        out_type=jax.ShapeDtypeStruct((B, D), jnp.float32),
        mesh=mesh,
        scratch_types=[
            pltpu.VMEM((b_per_w,), jnp.int32),
            pltpu.VMEM((NBUF, CHUNK, D), jnp.float32),
            pltpu.SemaphoreType.DMA((NBUF,)),
            pltpu.SemaphoreType.DMA((NBUF,)),
        ],
        compiler_params=pltpu.CompilerParams(use_tc_tiling_on_sc=False),
    )
    def gather_kernel(table_hbm, idx_hbm, out_hbm, idx_v, rows, gsem, wsem):
        wid = lax.axis_index("s") * 2 + lax.axis_index("c")
        base = wid * b_per_w
        pltpu.sync_copy(idx_hbm.at[pl.ds(base, b_per_w)], idx_v)

        def fire_gather(g, b):
            return pltpu.async_copy(
                table_hbm.at[idx_v.at[pl.ds(g * CHUNK, CHUNK)]],
                rows.at[b],
                gsem.at[b],
            )

        def fire_writeback(g, b, gather_desc):
            gather_desc.wait()
            pltpu.async_copy(
                rows.at[b],
                out_hbm.at[pl.ds(base + g * CHUNK, CHUNK)],
                wsem.at[b],
            )

        def wait_writeback(b):
            pltpu.make_async_copy(
                out_hbm.at[pl.ds(base, CHUNK)], rows.at[b], wsem.at[b]
            ).wait()

        descs = [fire_gather(b, b) for b in range(NBUF)]
        for b in range(NBUF):
            fire_writeback(b, b, descs[b])

        @pl.loop(1, n_groups)
        def _group(i):
            descs = []
            for b in range(NBUF):
                wait_writeback(b)
                descs.append(fire_gather(i * NBUF + b, b))
            for b in range(NBUF):
                fire_writeback(i * NBUF + b, b, descs[b])

        for b in range(NBUF):
            wait_writeback(b)

    return gather_kernel


def kernel(token_ids, embedding_matrix):
    S, T = token_ids.shape
    V, D = embedding_matrix.shape
    B = S * T
    flat_ids = token_ids.reshape(B).astype(jnp.int32)
    out = _build(B, V, D)(embedding_matrix, flat_ids)
    return out.reshape(S, T, D)


# speedup vs baseline: 1.5395x; 1.3872x over previous
import functools

import jax
import jax.numpy as jnp
from jax import lax
from jax.experimental import pallas as pl
from jax.experimental.pallas import tpu as pltpu
from jax.experimental.pallas import tpu_sc as plsc

NW = 32
SPW = 512
NT = 50
NB = SPW // 128
K = NT * NB


@functools.cache
def _build(S, T, V, D):
    mesh = plsc.VectorSubcoreMesh(core_axis_name="c", subcore_axis_name="s")

    @functools.partial(
        pl.kernel,
        out_type=jax.ShapeDtypeStruct((T, D // 8, S // 128, 8, 128), jnp.float32),
        mesh=mesh,
        scratch_types=[
            pltpu.VMEM((NT, SPW), jnp.int32),
            pltpu.VMEM((128, 32), jnp.float32),
            pltpu.VMEM((128, 32), jnp.float32),
            pltpu.VMEM((4, 8, 128), jnp.float32),
            pltpu.VMEM((4, 8, 128), jnp.float32),
            pltpu.SemaphoreType.DMA((2,)),
            pltpu.SemaphoreType.DMA((2,)),
        ],
        compiler_params=pltpu.CompilerParams(use_tc_tiling_on_sc=False, needs_layout_passes=False),
    )
    def k(tbl, iT, y5, idxv, rows0, rows1, tr0, tr1, gsem, wsem):
        wid = lax.axis_index("s") * 2 + lax.axis_index("c")
        s0 = wid * SPW
        pltpu.sync_copy(iT.at[:, pl.ds(s0, SPW)], idxv)
        rows = (rows0, rows1)
        tr = (tr0, tr1)
        iotas = [lax.iota(jnp.int32, 16) + (q * 16) for q in range(8)]

        def fire_gather(kk, p):
            t = kk // NB
            b = kk % NB
            pltpu.async_copy(
                tbl.at[idxv.at[t, pl.ds(b * 128, 128)]], rows[p], gsem.at[p]
            )

        def wait_gather(p):
            pltpu.make_async_copy(tbl.at[pl.ds(0, 128)], rows[p], gsem.at[p]).wait()

        def transpose(p):
            for d in range(32):
                col = jnp.full((16,), d, jnp.int32)
                for q in range(8):
                    v = plsc.load_gather(rows[p], [iotas[q], col])
                    tr[p][d // 8, d % 8, pl.ds(q * 16, 16)] = v

        def fire_out(kk, p):
            t = kk // NB
            b = kk % NB
            stg = wid * NB + b
            for dt in range(4):
                pltpu.async_copy(tr[p].at[dt], y5.at[t, dt, stg], wsem.at[p])

        def wait_out(p):
            for dt in range(4):
                pltpu.make_async_copy(y5.at[0, 0, 0], tr[p].at[dt], wsem.at[p]).wait()

        fire_gather(0, 0)
        fire_gather(1, 1)

        @pl.loop(0, K // 2)
        def _i(i):
            for p in range(2):
                kk = 2 * i + p
                wait_gather(p)

                @pl.when(i > 0)
                def _():
                    wait_out(p)

                transpose(p)

                @pl.when(kk + 2 < K)
                def _():
                    fire_gather(kk + 2, p)

                fire_out(kk, p)

        wait_out(0)
        wait_out(1)

    return k


def kernel(token_ids, embedding_matrix):
    S, T = token_ids.shape
    V, D = embedding_matrix.shape
    iT = token_ids.T.astype(jnp.int32)
    y5 = _build(S, T, V, D)(embedding_matrix, iT)
    return y5.transpose(2, 4, 0, 1, 3).reshape(S, T, D)


# speedup vs baseline: 1.8379x; 1.1938x over previous
import functools

import jax
import jax.numpy as jnp
from jax import lax
from jax.experimental import pallas as pl
from jax.experimental.pallas import tpu as pltpu
from jax.experimental.pallas import tpu_sc as plsc

NW = 32
SPW = 512
NT = 50
NB = SPW // 128
K = NT * NB


@functools.cache
def _build(S, T, V, D):
    mesh = plsc.VectorSubcoreMesh(core_axis_name="c", subcore_axis_name="s")

    @functools.partial(
        pl.kernel,
        out_type=jax.ShapeDtypeStruct((T, D // 8, S // 128, 8, 128), jnp.float32),
        mesh=mesh,
        scratch_types=[
            pltpu.VMEM((NT, SPW), jnp.int32),
            pltpu.VMEM((128, 32), jnp.float32),
            pltpu.VMEM((128, 32), jnp.float32),
            pltpu.VMEM((4, 8, 128), jnp.float32),
            pltpu.VMEM((4, 8, 128), jnp.float32),
            pltpu.SemaphoreType.DMA((2,)),
            pltpu.SemaphoreType.DMA((2,)),
        ],
        compiler_params=pltpu.CompilerParams(use_tc_tiling_on_sc=False, needs_layout_passes=False),
    )
    def k(tbl, iT, y5, idxv, rows0, rows1, tr0, tr1, gsem, wsem):
        wid = lax.axis_index("s") * 2 + lax.axis_index("c")
        s0 = wid * SPW
        pltpu.sync_copy(iT.at[:, pl.ds(s0, SPW)], idxv)
        rows = (rows0, rows1)
        tr = (tr0, tr1)
        iotas = [lax.iota(jnp.int32, 16) + (q * 16) for q in range(8)]

        def fire_gather(kk, p):
            t = kk // NB
            b = kk % NB
            pltpu.async_copy(
                tbl.at[idxv.at[t, pl.ds(b * 128, 128)]], rows[p], gsem.at[p]
            )

        def wait_gather(p):
            pltpu.make_async_copy(tbl.at[pl.ds(0, 128)], rows[p], gsem.at[p]).wait()

        def transpose(p):
            for dt in range(4):
                for ds_ in range(8):
                    d = dt * 8 + ds_
                    col = jnp.full((16,), d, jnp.int32)
                    vs = [plsc.load_gather(rows[p], [iotas[q], col]) for q in range(8)]
                    for q in range(8):
                        tr[p][dt, ds_, pl.ds(q * 16, 16)] = vs[q]

        def fire_out(kk, p):
            t = kk // NB
            b = kk % NB
            stg = wid * NB + b
            pltpu.async_copy(tr[p], y5.at[t, :, stg], wsem.at[p])

        def wait_out(p):
            pltpu.make_async_copy(y5.at[0, :, 0], tr[p], wsem.at[p]).wait()

        fire_gather(0, 0)
        fire_gather(1, 1)

        @pl.loop(0, K // 2)
        def _i(i):
            for p in range(2):
                kk = 2 * i + p
                wait_gather(p)

                @pl.when(i > 0)
                def _():
                    wait_out(p)

                transpose(p)

                @pl.when(kk + 2 < K)
                def _():
                    fire_gather(kk + 2, p)

                fire_out(kk, p)

        wait_out(0)
        wait_out(1)

    return k


def kernel(token_ids, embedding_matrix):
    S, T = token_ids.shape
    V, D = embedding_matrix.shape
    iT = token_ids.T.astype(jnp.int32)
    y5 = _build(S, T, V, D)(embedding_matrix, iT)
    return y5.transpose(2, 4, 0, 1, 3).reshape(S, T, D)
